# baseline (speedup 1.0000x reference)
import functools

import jax
import jax.numpy as jnp
from jax.experimental import pallas as pl
from jax.experimental.pallas import tpu as pltpu
from jax.experimental.pallas import tpu_sc as plsc

_BLK = 256
_GW = 16


def _routing(idx, n, num_experts, blk, m):
    e_range = jnp.arange(num_experts, dtype=jnp.int32)
    onehot = (idx[:, None] == e_range[None, :]).astype(jnp.int32)
    csum = jnp.cumsum(onehot, axis=0)
    counts = csum[-1]
    rank = jnp.take_along_axis(csum, idx[:, None], axis=1)[:, 0] - 1
    padded = ((counts + blk - 1) // blk) * blk
    ends = jnp.cumsum(padded)
    starts = ends - padded
    dst = starts[idx] + rank
    blk_base = jnp.arange(m // blk, dtype=jnp.int32) * blk
    block_expert = jnp.sum(
        (blk_base[:, None] >= ends[None, :]).astype(jnp.int32), axis=1)
    block_expert = jnp.minimum(block_expert, num_experts - 1)
    return dst, block_expert


def _sc_row_gather(table, idx):
    q = idx.shape[0]
    d = table.shape[1]
    mesh = plsc.VectorSubcoreMesh(core_axis_name="core",
                                  subcore_axis_name="subcore")
    num_workers = mesh.num_cores * mesh.num_subcores
    rows_per_w = q // num_workers
    nch = rows_per_w // _GW

    @functools.partial(
        pl.kernel,
        out_type=jax.ShapeDtypeStruct((q, d), table.dtype),
        mesh=mesh,
        scratch_types=[
            pltpu.VMEM((rows_per_w,), jnp.int32),
            pltpu.VMEM((_GW, d), table.dtype),
            pltpu.SemaphoreType.DMA,
        ],
    )
    def gather_kernel(x_hbm, i_hbm, o_hbm, idx_v, buf, sem):
        wid = (jax.lax.axis_index("subcore") * mesh.num_cores
               + jax.lax.axis_index("core"))
        base = wid * rows_per_w
        pltpu.sync_copy(i_hbm.at[pl.ds(base, rows_per_w)], idx_v)

        def chunk_body(c, carry):
            off = c * _GW
            pltpu.async_copy(
                x_hbm.at[idx_v.at[pl.ds(off, _GW)]], buf, sem).wait()
            pltpu.sync_copy(buf, o_hbm.at[pl.ds(base + off, _GW)])
            return carry

        jax.lax.fori_loop(0, nch, chunk_body, 0)

    return gather_kernel(table, idx)


def _sc_row_scatter(rows, idx, q, row_offset=0):
    d = rows.shape[1]
    nsc = idx.shape[0]
    mesh = plsc.VectorSubcoreMesh(core_axis_name="core",
                                  subcore_axis_name="subcore")
    num_workers = mesh.num_cores * mesh.num_subcores
    rows_per_w = nsc // num_workers
    nch = rows_per_w // _GW
    idx3d = idx.reshape((num_workers, nch, _GW))

    @functools.partial(
        pl.kernel,
        out_type=jax.ShapeDtypeStruct((q, d), rows.dtype),
        mesh=mesh,
        scratch_types=[
            pltpu.VMEM((nch, _GW), jnp.int32),
            pltpu.VMEM((_GW, d), rows.dtype),
            pltpu.SemaphoreType.DMA,
        ],
    )
    def scatter_kernel(x_hbm, i_hbm, o_hbm, idx_v, buf, sem):
        wid = (jax.lax.axis_index("subcore") * mesh.num_cores
               + jax.lax.axis_index("core"))
        base = row_offset + wid * rows_per_w
        pltpu.sync_copy(i_hbm.at[wid], idx_v)

        def chunk_body(c, carry):
            pltpu.sync_copy(x_hbm.at[pl.ds(base + c * _GW, _GW)], buf)
            pltpu.async_copy(buf, o_hbm.at[idx_v.at[c]], sem).wait()
            return carry

        jax.lax.fori_loop(0, nch, chunk_body, 0)

    return scatter_kernel(rows, idx3d)


def _sc_combine2(ys0, ys1, dst0, dst1, n):
    d = ys0.shape[1]
    half = n // 2
    mesh = plsc.VectorSubcoreMesh(core_axis_name="core",
                                  subcore_axis_name="subcore")
    num_workers = mesh.num_cores * mesh.num_subcores
    rows_per_w = half // num_workers
    nch = rows_per_w // _GW

    @functools.partial(
        pl.kernel,
        out_type=jax.ShapeDtypeStruct((n, d), ys0.dtype),
        mesh=mesh,
        scratch_types=[
            pltpu.VMEM((rows_per_w,), jnp.int32),
            pltpu.VMEM((_GW, d), ys0.dtype),
            pltpu.SemaphoreType.DMA,
        ],
    )
    def combine_kernel(y0_hbm, y1_hbm, i0_hbm, i1_hbm, o_hbm,
                       idx_v, buf, sem):
        wid = (jax.lax.axis_index("subcore") * mesh.num_cores
               + jax.lax.axis_index("core"))
        base = wid * rows_per_w

        def do_half(y_hbm, i_hbm, out_base):
            pltpu.sync_copy(i_hbm.at[pl.ds(base, rows_per_w)], idx_v)

            def chunk_body(c, carry):
                off = c * _GW
                pltpu.async_copy(
                    y_hbm.at[idx_v.at[pl.ds(off, _GW)]], buf, sem).wait()
                pltpu.sync_copy(buf, o_hbm.at[pl.ds(out_base + off, _GW)])
                return carry

            jax.lax.fori_loop(0, nch, chunk_body, 0)

        do_half(y0_hbm, i0_hbm, base)
        do_half(y1_hbm, i1_hbm, half + base)

    return combine_kernel(ys0, ys1, dst0, dst1)


def _mm_body(be_ref, xs_ref, w_ref, b_ref, o_ref, wbf_ref):
    i = pl.program_id(0)
    changed = jnp.logical_or(
        i == 0, be_ref[i] != be_ref[jnp.maximum(i - 1, 0)])

    @pl.when(changed)
    def _():
        wbf_ref[...] = w_ref[0].astype(jnp.bfloat16)

    acc = jnp.dot(xs_ref[...].astype(jnp.bfloat16), wbf_ref[...],
                  preferred_element_type=jnp.float32)
    o_ref[...] = jnp.maximum(acc + b_ref[0], 0.0)


def _expert_matmul(xs, w, b, block_expert, m, d):
    num_blocks = m // _BLK
    grid_spec = pltpu.PrefetchScalarGridSpec(
        num_scalar_prefetch=1,
        grid=(num_blocks,),
        in_specs=[
            pl.BlockSpec((_BLK, d), lambda i, be: (i, 0)),
            pl.BlockSpec((1, d, d), lambda i, be: (be[i], 0, 0)),
            pl.BlockSpec((1, 1, d), lambda i, be: (be[i], 0, 0)),
        ],
        out_specs=pl.BlockSpec((_BLK, d), lambda i, be: (i, 0)),
        scratch_shapes=[pltpu.VMEM((d, d), jnp.bfloat16)],
    )
    return pl.pallas_call(
        _mm_body,
        grid_spec=grid_spec,
        out_shape=jax.ShapeDtypeStruct((m, d), jnp.float32),
        compiler_params=pltpu.CompilerParams(
            dimension_semantics=("arbitrary",)),
    )(block_expert, xs, w, b.reshape(b.shape[0], 1, d))


def kernel(x, groups, W, b):
    n, d = x.shape
    num_experts = W.shape[0]
    half = n // 2
    m = half + num_experts * _BLK

    idx = groups[:, 0].astype(jnp.int32)
    dst0, be0 = _routing(idx[:half], half, num_experts, _BLK, m)
    dst1, be1 = _routing(idx[half:], half, num_experts, _BLK, m)

    xs0 = _sc_row_scatter(x, dst0, m)
    xs1 = _sc_row_scatter(x, dst1, m, row_offset=half)
    ys0 = _expert_matmul(xs0, W, b, be0, m, d)
    ys1 = _expert_matmul(xs1, W, b, be1, m, d)
    return _sc_combine2(ys0, ys1, dst0, dst1, n)


# speedup vs baseline: 1.2739x; 1.2739x over previous
import functools

import jax
import jax.numpy as jnp
from jax.experimental import pallas as pl
from jax.experimental.pallas import tpu as pltpu
from jax.experimental.pallas import tpu_sc as plsc

_BLK = 256
_GW = 16


def _routing(idx, n, num_experts, blk, m):
    e_range = jnp.arange(num_experts, dtype=jnp.int32)
    onehot = (idx[:, None] == e_range[None, :]).astype(jnp.int32)
    csum = jnp.cumsum(onehot, axis=0)
    counts = csum[-1]
    rank = jnp.take_along_axis(csum, idx[:, None], axis=1)[:, 0] - 1
    padded = ((counts + blk - 1) // blk) * blk
    ends = jnp.cumsum(padded)
    starts = ends - padded
    dst = starts[idx] + rank
    blk_base = jnp.arange(m // blk, dtype=jnp.int32) * blk
    block_expert = jnp.sum(
        (blk_base[:, None] >= ends[None, :]).astype(jnp.int32), axis=1)
    block_expert = jnp.minimum(block_expert, num_experts - 1)
    return dst, block_expert


def _sc_row_gather(table, idx):
    q = idx.shape[0]
    d = table.shape[1]
    mesh = plsc.VectorSubcoreMesh(core_axis_name="core",
                                  subcore_axis_name="subcore")
    num_workers = mesh.num_cores * mesh.num_subcores
    rows_per_w = q // num_workers
    nch = rows_per_w // _GW

    @functools.partial(
        pl.kernel,
        out_type=jax.ShapeDtypeStruct((q, d), table.dtype),
        mesh=mesh,
        scratch_types=[
            pltpu.VMEM((rows_per_w,), jnp.int32),
            pltpu.VMEM((_GW, d), table.dtype),
            pltpu.VMEM((_GW, d), table.dtype),
            pltpu.SemaphoreType.DMA,
            pltpu.SemaphoreType.DMA,
            pltpu.SemaphoreType.DMA,
            pltpu.SemaphoreType.DMA,
        ],
    )
    def gather_kernel(x_hbm, i_hbm, o_hbm, idx_v, buf0, buf1,
                      is0, is1, os0, os1):
        wid = (jax.lax.axis_index("subcore") * mesh.num_cores
               + jax.lax.axis_index("core"))
        base = wid * rows_per_w
        pltpu.sync_copy(i_hbm.at[pl.ds(base, rows_per_w)], idx_v)

        bufs, isems, osems = (buf0, buf1), (is0, is1), (os0, os1)

        def read(c):
            return pltpu.async_copy(
                x_hbm.at[idx_v.at[pl.ds(c * _GW, _GW)]],
                bufs[c % 2], isems[c % 2])

        def write(c):
            return pltpu.async_copy(
                bufs[c % 2], o_hbm.at[pl.ds(base + c * _GW, _GW)],
                osems[c % 2])

        reads, writes = {}, {}
        reads[0] = read(0)
        for c in range(nch):
            reads[c].wait()
            if c + 1 < nch:
                if c >= 1:
                    writes[c - 1].wait()
                reads[c + 1] = read(c + 1)
            writes[c] = write(c)
        if nch >= 2:
            writes[nch - 2].wait()
        writes[nch - 1].wait()

    return gather_kernel(table, idx)


def _sc_row_scatter(rows, idx, q, row_offset=0):
    d = rows.shape[1]
    nsc = idx.shape[0]
    mesh = plsc.VectorSubcoreMesh(core_axis_name="core",
                                  subcore_axis_name="subcore")
    num_workers = mesh.num_cores * mesh.num_subcores
    rows_per_w = nsc // num_workers
    nch = rows_per_w // _GW
    idx3d = idx.reshape((num_workers, nch, _GW))

    @functools.partial(
        pl.kernel,
        out_type=jax.ShapeDtypeStruct((q, d), rows.dtype),
        mesh=mesh,
        scratch_types=[
            pltpu.VMEM((nch, _GW), jnp.int32),
            pltpu.VMEM((_GW, d), rows.dtype),
            pltpu.VMEM((_GW, d), rows.dtype),
            pltpu.SemaphoreType.DMA,
            pltpu.SemaphoreType.DMA,
            pltpu.SemaphoreType.DMA,
            pltpu.SemaphoreType.DMA,
        ],
    )
    def scatter_kernel(x_hbm, i_hbm, o_hbm, idx_v, buf0, buf1,
                       is0, is1, os0, os1):
        wid = (jax.lax.axis_index("subcore") * mesh.num_cores
               + jax.lax.axis_index("core"))
        base = row_offset + wid * rows_per_w
        pltpu.sync_copy(i_hbm.at[wid], idx_v)

        bufs, isems, osems = (buf0, buf1), (is0, is1), (os0, os1)

        def read(c):
            return pltpu.async_copy(
                x_hbm.at[pl.ds(base + c * _GW, _GW)],
                bufs[c % 2], isems[c % 2])

        def write(c):
            return pltpu.async_copy(
                bufs[c % 2], o_hbm.at[idx_v.at[c]], osems[c % 2])

        reads, writes = {}, {}
        reads[0] = read(0)
        for c in range(nch):
            reads[c].wait()
            if c + 1 < nch:
                if c >= 1:
                    writes[c - 1].wait()
                reads[c + 1] = read(c + 1)
            writes[c] = write(c)
        if nch >= 2:
            writes[nch - 2].wait()
        writes[nch - 1].wait()

    return scatter_kernel(rows, idx3d)


def _mm_body(be_ref, xs_ref, w_ref, b_ref, o_ref, wbf_ref):
    i = pl.program_id(0)
    changed = jnp.logical_or(
        i == 0, be_ref[i] != be_ref[jnp.maximum(i - 1, 0)])

    @pl.when(changed)
    def _():
        wbf_ref[...] = w_ref[0].astype(jnp.bfloat16)

    acc = jnp.dot(xs_ref[...].astype(jnp.bfloat16), wbf_ref[...],
                  preferred_element_type=jnp.float32)
    o_ref[...] = jnp.maximum(acc + b_ref[0], 0.0)


def _expert_matmul(xs, w, b, block_expert, m, d):
    num_blocks = m // _BLK
    grid_spec = pltpu.PrefetchScalarGridSpec(
        num_scalar_prefetch=1,
        grid=(num_blocks,),
        in_specs=[
            pl.BlockSpec((_BLK, d), lambda i, be: (i, 0)),
            pl.BlockSpec((1, d, d), lambda i, be: (be[i], 0, 0)),
            pl.BlockSpec((1, 1, d), lambda i, be: (be[i], 0, 0)),
        ],
        out_specs=pl.BlockSpec((_BLK, d), lambda i, be: (i, 0)),
        scratch_shapes=[pltpu.VMEM((d, d), jnp.bfloat16)],
    )
    return pl.pallas_call(
        _mm_body,
        grid_spec=grid_spec,
        out_shape=jax.ShapeDtypeStruct((m, d), jnp.float32),
        compiler_params=pltpu.CompilerParams(
            dimension_semantics=("arbitrary",)),
    )(block_expert, xs, w, b.reshape(b.shape[0], 1, d))


def kernel(x, groups, W, b):
    n, d = x.shape
    num_experts = W.shape[0]
    m = n + num_experts * _BLK

    idx = groups[:, 0].astype(jnp.int32)
    dst, block_expert = _routing(idx, n, num_experts, _BLK, m)

    xs = _sc_row_scatter(x, dst, m)
    ys = _expert_matmul(xs, W, b, block_expert, m, d)
    return _sc_row_gather(ys, dst)
